# baseline (speedup 1.0000x reference)
import functools

import jax
import jax.numpy as jnp
from jax import lax
from jax.experimental import pallas as pl
from jax.experimental.pallas import tpu as pltpu
from jax.experimental.pallas import tpu_sc as plsc

N = 10000
E = 160000
D = 256
DE = 16

NS = 16
EPT = E // NS
KE = 80
NCHE = EPT // KE
KS = 400
NCHS = EPT // KS
NPAIR = 12
NB = 1000


def _tc1_body(x_ref, wa_in_ref, wa_out_ref, w4_ref, wei_ref, weo_ref,
              a_in_ref, a_out_ref, scal_ref, *xw_refs):
    xb = x_ref[...]
    a_in = xb @ wa_in_ref[...]
    a_out = xb @ wa_out_ref[...]
    a_in_ref[...] = a_in
    a_out_ref[...] = a_out
    scal_ref[...] = jnp.concatenate([a_in, a_out], axis=1) @ w4_ref[...]
    xwi = xb @ wei_ref[...]
    xwo = xb @ weo_ref[...]
    for q in range(8):
        xw_refs[q][...] = xwi[:, 32 * q:32 * (q + 1)]
        xw_refs[8 + q][...] = xwo[:, 32 * q:32 * (q + 1)]


def _tc1(x, wa_in, wa_out, w4, wei, weo):
    nb = N // NB
    f32 = jnp.float32
    return pl.pallas_call(
        _tc1_body,
        grid=(nb,),
        in_specs=[
            pl.BlockSpec((NB, D), lambda i: (i, 0)),
            pl.BlockSpec((D, 32), lambda i: (0, 0)),
            pl.BlockSpec((D, 32), lambda i: (0, 0)),
            pl.BlockSpec((64, 4), lambda i: (0, 0)),
            pl.BlockSpec((D, D), lambda i: (0, 0)),
            pl.BlockSpec((D, D), lambda i: (0, 0)),
        ],
        out_specs=[
            pl.BlockSpec((NB, 32), lambda i: (i, 0)),
            pl.BlockSpec((NB, 32), lambda i: (i, 0)),
            pl.BlockSpec((NB, 4), lambda i: (i, 0)),
        ] + [pl.BlockSpec((NB, 32), lambda i: (i, 0)) for _ in range(16)],
        out_shape=[
            jax.ShapeDtypeStruct((N, 32), f32),
            jax.ShapeDtypeStruct((N, 32), f32),
            jax.ShapeDtypeStruct((N, 4), f32),
        ] + [jax.ShapeDtypeStruct((N, 32), f32) for _ in range(16)],
    )(x, wa_in, wa_out, w4, wei, weo)


def _tc2_body(ea_ref, w16_ref, w1_ref, c_ref, cs_ref):
    cb = ea_ref[...] @ w16_ref[0]
    c_ref[...] = cb
    cs_ref[...] = cb @ w1_ref[0]


def _tc2(edge_attr_x, w16s, w1s):
    be = 2000
    nb = (2 * E) // be
    return pl.pallas_call(
        _tc2_body,
        grid=(nb,),
        in_specs=[
            pl.BlockSpec((be, DE), lambda i: (i, 0)),
            pl.BlockSpec((1, DE, DE), lambda i: (i // (nb // 2), 0, 0)),
            pl.BlockSpec((1, DE, 1), lambda i: (i // (nb // 2), 0, 0)),
        ],
        out_specs=[
            pl.BlockSpec((be, DE), lambda i: (i, 0)),
            pl.BlockSpec((be, 1), lambda i: (i, 0)),
        ],
        out_shape=[
            jax.ShapeDtypeStruct((2 * E, DE), jnp.float32),
            jax.ShapeDtypeStruct((2 * E, 1), jnp.float32),
        ],
    )(edge_attr_x, w16s, w1s)


def _sc_body(dircid, eidx, a_t, scal_d, c_h, cs_h, *rest):
    tbls = rest[:8]
    zrows, ea_o, xacc_o = rest[8], rest[9], rest[10]
    (scal_v, w_v, idx1_all, idx2_all, ga1, ga2, c_v, cs_v, ea_v0, ea_v1,
     xwb0, xwb1, accum, sem0, sem1, sem2, sem3, sem4, sem5) = rest[11:]
    sid = lax.axis_index("s")

    if True:
        eoff = dircid * E
        tbase = sid * EPT
        pltpu.sync_copy(scal_d, scal_v)

        pltpu.sync_copy(eidx.at[pl.ds(eoff + tbase, EPT)], idx1_all)
        pltpu.sync_copy(eidx.at[pl.ds((E - eoff) + tbase, EPT)], idx2_all)

        def make_eb(ea_v, st_sem):
            def eb(i, _):
                base_l = i * KE
                base_g = tbase + base_l
                sl_e = pl.ds(base_l, KE)
                d1 = pltpu.async_copy(
                    c_h.at[pl.ds(eoff + base_g, KE)], c_v, sem0)
                d2 = pltpu.async_copy(
                    cs_h.at[pl.ds(eoff + base_g, KE)], cs_v, sem1)
                d3 = pltpu.async_copy(a_t.at[idx1_all.at[sl_e]], ga1, sem2)
                d4 = pltpu.async_copy(a_t.at[idx2_all.at[sl_e]], ga2, sem3)
                d1.wait()
                d2.wait()
                d3.wait()
                d4.wait()
                for g in range(KE // 16):
                    sl = pl.ds(base_l + g * 16, 16)
                    z = (plsc.load_gather(scal_v, [idx1_all[sl] * 2])
                         + plsc.load_gather(scal_v, [idx2_all[sl] * 2 + 1])
                         + cs_v[pl.ds(g * 16, 16)])
                    w_v[sl] = 1.0 / (1.0 + jnp.exp(-z))
                @pl.when(i >= 2)
                def _():
                    pltpu.make_async_copy(
                        ea_v, ea_o.at[pl.ds(tbase, KE)], st_sem).wait()
                for k in range(KE):
                    ea_v[k, :] = ga1[k, 0:16] + ga2[k, 16:32] + c_v[k, :]
                pltpu.async_copy(
                    ea_v, ea_o.at[pl.ds(base_g, KE)], st_sem)
                return 0
            return eb

        eb0 = make_eb(ea_v0, sem4)
        eb1 = make_eb(ea_v1, sem5)

        def ebpair(p, _):
            eb0(2 * p, 0)
            eb1(2 * p + 1, 0)
            return 0

        lax.fori_loop(0, NCHE // 2, ebpair, 0)
        eb0(NCHE - 1, 0)
        pltpu.make_async_copy(
            ea_v0, ea_o.at[pl.ds(tbase, KE)], sem4).wait()
        pltpu.make_async_copy(
            ea_v1, ea_o.at[pl.ds(tbase, KE)], sem5).wait()

        def gather_start(tbl, i, buf, sem):
            pltpu.async_copy(tbl.at[idx1_all.at[pl.ds(i * KS, KS)]], buf, sem)

        def gather_wait(tbl, buf, sem):
            pltpu.make_async_copy(tbl.at[pl.ds(0, KS)], buf, sem).wait()

        def scale(i, buf):
            base_w = i * KS

            def gb(g, _):
                w16 = w_v[pl.ds(base_w + g * 16, 16)]
                for j in range(16):
                    wk = w16.at[jnp.full((16,), j, jnp.int32)].get(
                        mode="promise_in_bounds")
                    k = g * 16 + j
                    for c in range(2):
                        csl = pl.ds(c * 16, 16)
                        buf[k, csl] = buf[k, csl] * wk
                return 0

            lax.fori_loop(0, KS // 16, gb, 0)

        def scatter_start(i, buf, sem):
            idx = idx2_all.at[pl.ds(i * KS, KS)]
            pltpu.async_copy(buf, accum.at[idx], sem, add=True)

        def scatter_wait(buf, sem):
            idx = idx2_all.at[pl.ds(0, KS)]
            pltpu.make_async_copy(buf, accum.at[idx], sem).wait()

        for h, tbl in enumerate(tbls):
            pltpu.sync_copy(zrows, accum.at[pl.ds(sid * (N // NS), N // NS)])
            plsc.subcore_barrier()

            gather_start(tbl, 0, xwb0, sem0)

            def pair(p, _):
                i0 = 2 * p

                @pl.when(p > 0)
                def _():
                    scatter_wait(xwb1, sem3)

                gather_start(tbl, i0 + 1, xwb1, sem1)
                gather_wait(tbl, xwb0, sem0)
                scale(i0, xwb0)
                scatter_start(i0, xwb0, sem2)
                gather_wait(tbl, xwb1, sem1)
                scale(i0 + 1, xwb1)
                scatter_wait(xwb0, sem2)
                gather_start(tbl, i0 + 2, xwb0, sem0)
                scatter_start(i0 + 1, xwb1, sem3)
                return 0

            lax.fori_loop(0, NPAIR, pair, 0)
            scatter_wait(xwb1, sem3)
            gather_wait(tbl, xwb0, sem0)
            scale(NCHS - 1, xwb0)
            scatter_start(NCHS - 1, xwb0, sem2)
            scatter_wait(xwb0, sem2)
            plsc.subcore_barrier()

            pltpu.sync_copy(
                accum.at[pl.ds(sid * (N // NS), N // NS)],
                xacc_o.at[pl.ds(h * N + sid * (N // NS), N // NS)])
            plsc.subcore_barrier()


def _sc_call(eidx, a_in, a_out, scal_in, scal_out, c, cs, xws, zrows):
    f32 = jnp.float32

    def one(dircid, a_t, scal_d, tbls):
        mesh = plsc.VectorSubcoreMesh(
            core_axis_name="c", subcore_axis_name="s", num_cores=1)
        kfn = pl.kernel(
            functools.partial(_sc_body, dircid),
            out_type=[
                jax.ShapeDtypeStruct((E, DE), f32),
                jax.ShapeDtypeStruct((8 * N, 32), f32),
            ],
            mesh=mesh,
            compiler_params=pltpu.CompilerParams(
                needs_layout_passes=False, use_tc_tiling_on_sc=False),
            scratch_types=[
                pltpu.VMEM((N * 2,), f32),
                pltpu.VMEM((EPT,), f32),
                pltpu.VMEM((EPT,), jnp.int32),
                pltpu.VMEM((EPT,), jnp.int32),
                pltpu.VMEM((KE, 32), f32),
                pltpu.VMEM((KE, 32), f32),
                pltpu.VMEM((KE, DE), f32),
                pltpu.VMEM((KE,), f32),
                pltpu.VMEM((KE, DE), f32),
                pltpu.VMEM((KE, DE), f32),
                pltpu.VMEM((KS, 32), f32),
                pltpu.VMEM((KS, 32), f32),
                pltpu.VMEM_SHARED((N, 32), f32),
            ] + [pltpu.SemaphoreType.DMA] * 6,
        )
        return kfn(eidx, a_t, scal_d, c, cs, *tbls, zrows)

    ea_i, xacc_i = one(0, a_in, scal_in, xws[:8])
    ea_t, xacc_t = one(1, a_out, scal_out, xws[8:])
    return ea_i, ea_t, xacc_i, xacc_t


def _tc3_body(x_ref, *refs):
    acc_refs = refs[:8]
    b_ref, o_ref = refs[8], refs[9]
    xin = jnp.concatenate([r[...] for r in acc_refs[:4]], axis=1)
    xout = jnp.concatenate([r[...] for r in acc_refs[4:]], axis=1)
    o_ref[...] = x_ref[...] + xin + xout + b_ref[0]


def _tc3(x, xacc_i, xacc_t, b2):
    nb = N // NB
    return pl.pallas_call(
        _tc3_body,
        grid=(nb, 2),
        in_specs=[pl.BlockSpec((NB, 128), lambda i, j: (i, j))] + [
            pl.BlockSpec(
                (NB, 32),
                functools.partial(
                    lambda s, i, j: ((4 * j + s) * (N // NB) + i, 0), q % 4))
            for q in range(8)
        ] + [
            pl.BlockSpec((1, 1, 128), lambda i, j: (j, 0, 0)),
        ],
        out_specs=pl.BlockSpec((NB, 128), lambda i, j: (i, j)),
        out_shape=jax.ShapeDtypeStruct((N, D), jnp.float32),
    )(x, *([xacc_i] * 4), *([xacc_t] * 4), b2)


def kernel(x, edge_attr_x, edge_index, W_in_0, W_in_1, W_out_0, W_out_1,
           W_emb_in, b_emb_in, W_emb_out, b_emb_out):
    f32 = jnp.float32
    wa_in = jnp.concatenate([W_in_0[:D], W_in_0[D + DE:]], axis=1)
    wa_out = jnp.concatenate([W_out_0[:D], W_out_0[D + DE:]], axis=1)
    w16s = jnp.stack([W_in_0[D:D + DE], W_out_0[D:D + DE]])
    w1s = jnp.stack([W_in_1, W_out_1])
    w4 = jnp.zeros((64, 4), f32)
    w4 = w4.at[0:16, 0].set(W_in_1[:, 0]).at[16:32, 1].set(W_in_1[:, 0])
    w4 = w4.at[32:48, 2].set(W_out_1[:, 0]).at[48:64, 3].set(W_out_1[:, 0])
    b2 = (b_emb_in + b_emb_out).reshape(2, 1, 128)
    zrows = jnp.zeros((N // NS, 32), f32)
    eidx = edge_index.reshape(2 * E).astype(jnp.int32)

    a_in, a_out, scal, *xws = _tc1(x, wa_in, wa_out, w4, W_emb_in, W_emb_out)
    scal_in = scal[:, :2].reshape(2 * N)
    scal_out = scal[:, 2:].reshape(2 * N)
    c, cs = _tc2(edge_attr_x, w16s, w1s)
    ea_i, ea_t, xacc_i, xacc_t = _sc_call(
        eidx, a_in, a_out, scal_in, scal_out, c, cs.reshape(2 * E),
        xws, zrows)
    x_new = _tc3(x, xacc_i, xacc_t, b2)
    return (x_new, jnp.concatenate([ea_i, ea_t], axis=0))


# speedup vs baseline: 1.3045x; 1.3045x over previous
import functools

import jax
import jax.numpy as jnp
from jax import lax
from jax.experimental import pallas as pl
from jax.experimental.pallas import tpu as pltpu
from jax.experimental.pallas import tpu_sc as plsc

N = 10000
E = 160000
D = 256
DE = 16

NS = 16
EPT = E // NS
KE = 80
NCHE = EPT // KE
KS = 400
NCHS = EPT // KS
NPAIR = 12
NB = 1000


def _tc1_body(x_ref, wa_in_ref, wa_out_ref, w4_ref, wei_ref, weo_ref,
              a_in_ref, a_out_ref, scal_ref, *xw_refs):
    xb = x_ref[...]
    a_in = xb @ wa_in_ref[...]
    a_out = xb @ wa_out_ref[...]
    a_in_ref[...] = a_in
    a_out_ref[...] = a_out
    scal_ref[...] = jnp.concatenate([a_in, a_out], axis=1) @ w4_ref[...]
    xwi = xb @ wei_ref[...]
    xwo = xb @ weo_ref[...]
    for q in range(8):
        xw_refs[q][...] = xwi[:, 32 * q:32 * (q + 1)]
        xw_refs[8 + q][...] = xwo[:, 32 * q:32 * (q + 1)]


def _tc1(x, wa_in, wa_out, w4, wei, weo):
    nb = N // NB
    f32 = jnp.float32
    return pl.pallas_call(
        _tc1_body,
        grid=(nb,),
        in_specs=[
            pl.BlockSpec((NB, D), lambda i: (i, 0)),
            pl.BlockSpec((D, 32), lambda i: (0, 0)),
            pl.BlockSpec((D, 32), lambda i: (0, 0)),
            pl.BlockSpec((64, 4), lambda i: (0, 0)),
            pl.BlockSpec((D, D), lambda i: (0, 0)),
            pl.BlockSpec((D, D), lambda i: (0, 0)),
        ],
        out_specs=[
            pl.BlockSpec((NB, 32), lambda i: (i, 0)),
            pl.BlockSpec((NB, 32), lambda i: (i, 0)),
            pl.BlockSpec((NB, 4), lambda i: (i, 0)),
        ] + [pl.BlockSpec((NB, 32), lambda i: (i, 0)) for _ in range(16)],
        out_shape=[
            jax.ShapeDtypeStruct((N, 32), f32),
            jax.ShapeDtypeStruct((N, 32), f32),
            jax.ShapeDtypeStruct((N, 4), f32),
        ] + [jax.ShapeDtypeStruct((N, 32), f32) for _ in range(16)],
    )(x, wa_in, wa_out, w4, wei, weo)


def _tc2_body(ea_ref, w16_ref, w1_ref, c_ref, cs_ref):
    cb = ea_ref[...] @ w16_ref[0]
    c_ref[...] = cb
    cs_ref[...] = cb @ w1_ref[0]


def _tc2(edge_attr_x, w16s, w1s):
    be = 2000
    nb = (2 * E) // be
    return pl.pallas_call(
        _tc2_body,
        grid=(nb,),
        in_specs=[
            pl.BlockSpec((be, DE), lambda i: (i, 0)),
            pl.BlockSpec((1, DE, DE), lambda i: (i // (nb // 2), 0, 0)),
            pl.BlockSpec((1, DE, 1), lambda i: (i // (nb // 2), 0, 0)),
        ],
        out_specs=[
            pl.BlockSpec((be, DE), lambda i: (i, 0)),
            pl.BlockSpec((be, 1), lambda i: (i, 0)),
        ],
        out_shape=[
            jax.ShapeDtypeStruct((2 * E, DE), jnp.float32),
            jax.ShapeDtypeStruct((2 * E, 1), jnp.float32),
        ],
    )(edge_attr_x, w16s, w1s)


def _sc_body(eidx, a_in, a_out, scal_in_h, scal_out_h, c_h, cs_h, *rest):
    xw_tbls = rest[:16]
    zrows, ea_o, xacc_o = rest[16], rest[17], rest[18]
    (scal_v, w_v, idx1_all, idx2_all, ga1, ga2, c_v, cs_v, ea_v0, ea_v1,
     xwb0, xwb1, accum, sem0, sem1, sem2, sem3, sem4, sem5) = rest[19:]
    cid = lax.axis_index("c")
    sid = lax.axis_index("s")

    def run(dircid, a_t, scal_d, tbls):
        eoff = dircid * E
        tbase = sid * EPT
        pltpu.sync_copy(scal_d, scal_v)

        pltpu.sync_copy(eidx.at[pl.ds(eoff + tbase, EPT)], idx1_all)
        pltpu.sync_copy(eidx.at[pl.ds((E - eoff) + tbase, EPT)], idx2_all)

        def make_eb(ea_v, st_sem):
            def eb(i, _):
                base_l = i * KE
                base_g = tbase + base_l
                sl_e = pl.ds(base_l, KE)
                d1 = pltpu.async_copy(
                    c_h.at[pl.ds(eoff + base_g, KE)], c_v, sem0)
                d2 = pltpu.async_copy(
                    cs_h.at[pl.ds(eoff + base_g, KE)], cs_v, sem1)
                d3 = pltpu.async_copy(a_t.at[idx1_all.at[sl_e]], ga1, sem2)
                d4 = pltpu.async_copy(a_t.at[idx2_all.at[sl_e]], ga2, sem3)
                d1.wait()
                d2.wait()
                d3.wait()
                d4.wait()
                for g in range(KE // 16):
                    sl = pl.ds(base_l + g * 16, 16)
                    z = (plsc.load_gather(scal_v, [idx1_all[sl] * 2])
                         + plsc.load_gather(scal_v, [idx2_all[sl] * 2 + 1])
                         + cs_v[pl.ds(g * 16, 16)])
                    w_v[sl] = 1.0 / (1.0 + jnp.exp(-z))
                @pl.when(i >= 2)
                def _():
                    pltpu.make_async_copy(
                        ea_v, ea_o.at[pl.ds(eoff + tbase, KE)], st_sem).wait()
                for k in range(KE):
                    ea_v[k, :] = ga1[k, 0:16] + ga2[k, 16:32] + c_v[k, :]
                pltpu.async_copy(
                    ea_v, ea_o.at[pl.ds(eoff + base_g, KE)], st_sem)
                return 0
            return eb

        eb0 = make_eb(ea_v0, sem4)
        eb1 = make_eb(ea_v1, sem5)

        def ebpair(p, _):
            eb0(2 * p, 0)
            eb1(2 * p + 1, 0)
            return 0

        lax.fori_loop(0, NCHE // 2, ebpair, 0)
        eb0(NCHE - 1, 0)
        pltpu.make_async_copy(
            ea_v0, ea_o.at[pl.ds(eoff + tbase, KE)], sem4).wait()
        pltpu.make_async_copy(
            ea_v1, ea_o.at[pl.ds(eoff + tbase, KE)], sem5).wait()

        def gather_start(tbl, i, buf, sem):
            pltpu.async_copy(tbl.at[idx1_all.at[pl.ds(i * KS, KS)]], buf, sem)

        def gather_wait(tbl, buf, sem):
            pltpu.make_async_copy(tbl.at[pl.ds(0, KS)], buf, sem).wait()

        def scale(i, buf):
            base_w = i * KS

            def gb(g, _):
                w16 = w_v[pl.ds(base_w + g * 16, 16)]
                for j in range(16):
                    wk = w16.at[jnp.full((16,), j, jnp.int32)].get(
                        mode="promise_in_bounds")
                    k = g * 16 + j
                    for c in range(2):
                        csl = pl.ds(c * 16, 16)
                        buf[k, csl] = buf[k, csl] * wk
                return 0

            lax.fori_loop(0, KS // 16, gb, 0)

        def scatter_start(i, buf, sem):
            idx = idx2_all.at[pl.ds(i * KS, KS)]
            pltpu.async_copy(buf, accum.at[idx], sem, add=True)

        def scatter_wait(buf, sem):
            idx = idx2_all.at[pl.ds(0, KS)]
            pltpu.make_async_copy(buf, accum.at[idx], sem).wait()

        for h, tbl in enumerate(tbls):
            pltpu.sync_copy(zrows, accum.at[pl.ds(sid * (N // NS), N // NS)])
            plsc.subcore_barrier()

            gather_start(tbl, 0, xwb0, sem0)

            def pair(p, _):
                i0 = 2 * p

                @pl.when(p > 0)
                def _():
                    scatter_wait(xwb1, sem3)

                gather_start(tbl, i0 + 1, xwb1, sem1)
                gather_wait(tbl, xwb0, sem0)
                scale(i0, xwb0)
                scatter_start(i0, xwb0, sem2)
                gather_wait(tbl, xwb1, sem1)
                scale(i0 + 1, xwb1)
                scatter_wait(xwb0, sem2)
                gather_start(tbl, i0 + 2, xwb0, sem0)
                scatter_start(i0 + 1, xwb1, sem3)
                return 0

            lax.fori_loop(0, NPAIR, pair, 0)
            scatter_wait(xwb1, sem3)
            gather_wait(tbl, xwb0, sem0)
            scale(NCHS - 1, xwb0)
            scatter_start(NCHS - 1, xwb0, sem2)
            scatter_wait(xwb0, sem2)
            plsc.subcore_barrier()

            pltpu.sync_copy(
                accum.at[pl.ds(sid * (N // NS), N // NS)],
                xacc_o.at[pl.ds(
                    (8 * dircid + h) * N + sid * (N // NS), N // NS)])
            plsc.subcore_barrier()


    @pl.when(cid == 0)
    def _():
        run(0, a_in, scal_in_h, xw_tbls[:8])

    @pl.when(cid == 1)
    def _():
        run(1, a_out, scal_out_h, xw_tbls[8:])


def _sc_call(eidx, a_in, a_out, scal_in, scal_out, c, cs, xws, zrows):
    f32 = jnp.float32
    mesh = plsc.VectorSubcoreMesh(core_axis_name="c", subcore_axis_name="s")
    kfn = pl.kernel(
        _sc_body,
        out_type=[
            jax.ShapeDtypeStruct((2 * E, DE), f32),
            jax.ShapeDtypeStruct((16 * N, 32), f32),
        ],
        mesh=mesh,
        compiler_params=pltpu.CompilerParams(
            needs_layout_passes=False, use_tc_tiling_on_sc=False),
        scratch_types=[
            pltpu.VMEM((N * 2,), f32),
            pltpu.VMEM((EPT,), f32),
            pltpu.VMEM((EPT,), jnp.int32),
            pltpu.VMEM((EPT,), jnp.int32),
            pltpu.VMEM((KE, 32), f32),
            pltpu.VMEM((KE, 32), f32),
            pltpu.VMEM((KE, DE), f32),
            pltpu.VMEM((KE,), f32),
            pltpu.VMEM((KE, DE), f32),
            pltpu.VMEM((KE, DE), f32),
            pltpu.VMEM((KS, 32), f32),
            pltpu.VMEM((KS, 32), f32),
            pltpu.VMEM_SHARED((N, 32), f32),
        ] + [pltpu.SemaphoreType.DMA] * 6,
    )
    return kfn(eidx, a_in, a_out, scal_in, scal_out, c, cs, *xws, zrows)


def _tc3_body(x_ref, *refs):
    acc_refs = refs[:8]
    b_ref, o_ref = refs[8], refs[9]
    xin = jnp.concatenate([r[...] for r in acc_refs[:4]], axis=1)
    xout = jnp.concatenate([r[...] for r in acc_refs[4:]], axis=1)
    o_ref[...] = x_ref[...] + xin + xout + b_ref[0]


def _tc3(x, xacc, b2):
    nb = N // NB
    return pl.pallas_call(
        _tc3_body,
        grid=(nb, 2),
        in_specs=[pl.BlockSpec((NB, 128), lambda i, j: (i, j))] + [
            pl.BlockSpec(
                (NB, 32),
                functools.partial(
                    lambda d, s, i, j: ((8 * d + 4 * j + s) * (N // NB) + i, 0),
                    q // 4, q % 4))
            for q in range(8)
        ] + [
            pl.BlockSpec((1, 1, 128), lambda i, j: (j, 0, 0)),
        ],
        out_specs=pl.BlockSpec((NB, 128), lambda i, j: (i, j)),
        out_shape=jax.ShapeDtypeStruct((N, D), jnp.float32),
    )(x, *([xacc] * 8), b2)


def kernel(x, edge_attr_x, edge_index, W_in_0, W_in_1, W_out_0, W_out_1,
           W_emb_in, b_emb_in, W_emb_out, b_emb_out):
    f32 = jnp.float32
    wa_in = jnp.concatenate([W_in_0[:D], W_in_0[D + DE:]], axis=1)
    wa_out = jnp.concatenate([W_out_0[:D], W_out_0[D + DE:]], axis=1)
    w16s = jnp.stack([W_in_0[D:D + DE], W_out_0[D:D + DE]])
    w1s = jnp.stack([W_in_1, W_out_1])
    w4 = jnp.zeros((64, 4), f32)
    w4 = w4.at[0:16, 0].set(W_in_1[:, 0]).at[16:32, 1].set(W_in_1[:, 0])
    w4 = w4.at[32:48, 2].set(W_out_1[:, 0]).at[48:64, 3].set(W_out_1[:, 0])
    b2 = (b_emb_in + b_emb_out).reshape(2, 1, 128)
    zrows = jnp.zeros((N // NS, 32), f32)
    eidx = edge_index.reshape(2 * E).astype(jnp.int32)

    a_in, a_out, scal, *xws = _tc1(x, wa_in, wa_out, w4, W_emb_in, W_emb_out)
    scal_in = scal[:, :2].reshape(2 * N)
    scal_out = scal[:, 2:].reshape(2 * N)
    c, cs = _tc2(edge_attr_x, w16s, w1s)
    ea, xacc = _sc_call(eidx, a_in, a_out, scal_in, scal_out, c,
                        cs.reshape(2 * E), xws, zrows)
    x_new = _tc3(x, xacc, b2)
    return (x_new, ea)
